# baseline (speedup 1.0000x reference)
import functools

import jax
import jax.numpy as jnp
from jax import lax
from jax.experimental import pallas as pl
from jax.experimental.pallas import tpu as pltpu
from jax.experimental.pallas import tpu_sc as plsc

NC = 2
NS = 16
NW = NC * NS
CH = 128
CPBT = 160
SPB0 = 80
SPB1 = 80


def _sc_agg_rows_body(n_pad, d, xp, srcm, dstm,
                      out_s, out_cnt,
                      srcba, srcbb, dstba, dstbb, bufa, bufb, cnt2d,
                      semsa, semsb, semda, semdb, semga, semgb, semca, semcb,
                      acc):
    sid = lax.axis_index("s")
    rpt = n_pad // NS
    rows0 = sid * rpt

    def _zb(t, carry):
        bufa[t >> 3, pl.ds((t & 7) * 16, 16)] = jnp.zeros((16,), jnp.float32)
        return carry
    lax.fori_loop(0, CH * (d // 16), _zb, 0)
    for i in range(rpt // CH):
        pltpu.sync_copy(bufa, acc.at[pl.ds(rows0 + i * CH, CH)])

    def _zc(t, carry):
        cnt2d[t >> 3, pl.ds((t & 7) * 16, 16)] = jnp.zeros((16,), jnp.float32)
        return carry
    lax.fori_loop(0, (n_pad // CH) * (CH // 16), _zc, 0)

    cpb = CPBT
    c0 = sid * CPBT
    plsc.subcore_barrier()

    pltpu.async_copy(srcm.at[c0], srcba, semsa)
    pltpu.async_copy(dstm.at[c0], dstba, semda)
    pltpu.async_copy(srcm.at[c0 + 1], srcbb, semsb)
    pltpu.async_copy(dstm.at[c0 + 1], dstbb, semdb)
    pltpu.make_async_copy(srcm.at[c0], srcba, semsa).wait()
    pltpu.async_copy(xp.at[srcba], bufa, semga)

    def _count(dstb):
        def _cs(k, carry):
            dv = dstb[pl.ds(k * 16, 16)]
            occ, last = plsc.scan_count(dv)
            vals = occ.astype(jnp.float32)
            plsc.addupdate_scatter(
                cnt2d,
                [lax.shift_right_logical(dv, 7), lax.bitwise_and(dv, 127)],
                vals, mask=last)
            return carry
        lax.fori_loop(0, CH // 16, _cs, 0)

    def _step(j, carry):
        ia = 2 * j
        ib = 2 * j + 1
        pltpu.make_async_copy(srcm.at[c0 + ib], srcbb, semsb).wait()
        pltpu.async_copy(xp.at[srcbb], bufb, semgb)
        pltpu.make_async_copy(xp.at[srcba], bufa, semga).wait()
        pltpu.make_async_copy(dstm.at[c0 + ia], dstba, semda).wait()
        pltpu.async_copy(bufa, acc.at[dstba], semca, add=True)
        _count(dstba)
        pltpu.make_async_copy(bufa, acc.at[dstba], semca).wait()

        @pl.when(ia + 2 < cpb)
        def _():
            pltpu.async_copy(srcm.at[c0 + ia + 2], srcba, semsa)
            pltpu.async_copy(dstm.at[c0 + ia + 2], dstba, semda)
            pltpu.make_async_copy(srcm.at[c0 + ia + 2], srcba, semsa).wait()
            pltpu.async_copy(xp.at[srcba], bufa, semga)

        pltpu.make_async_copy(xp.at[srcbb], bufb, semgb).wait()
        pltpu.make_async_copy(dstm.at[c0 + ib], dstbb, semdb).wait()
        pltpu.async_copy(bufb, acc.at[dstbb], semcb, add=True)
        _count(dstbb)
        pltpu.make_async_copy(bufb, acc.at[dstbb], semcb).wait()

        @pl.when(ib + 2 < cpb)
        def _():
            pltpu.async_copy(srcm.at[c0 + ib + 2], srcbb, semsb)
            pltpu.async_copy(dstm.at[c0 + ib + 2], dstbb, semdb)

        return carry

    lax.fori_loop(0, cpb // 2, _step, 0)
    plsc.subcore_barrier()

    for i in range(rpt // CH):
        buf = bufa if i % 2 == 0 else bufb
        pltpu.sync_copy(acc.at[pl.ds(rows0 + i * CH, CH)], buf)
        pltpu.sync_copy(buf, out_s.at[pl.ds(rows0 + i * CH, CH)])
    pltpu.sync_copy(cnt2d, out_cnt.at[sid])


def _sc_agg_scalar_body(n_pad, u, srcm, dstm, out0, out1,
                        uloc, srcba, srcbb, dstba, dstbb, valba, valbb,
                        semsa, semsb, semda, semdb, semca, semcb, acc):
    cid = lax.axis_index("c")
    sid = lax.axis_index("s")
    rpt = n_pad // NS
    rows0 = sid * rpt

    def _zv(k, carry):
        valba[pl.ds(k * 16, 16)] = jnp.zeros((16,), jnp.float32)
        return carry
    lax.fori_loop(0, CH // 16, _zv, 0)
    for i in range(rpt // CH):
        pltpu.sync_copy(valba, acc.at[pl.ds(rows0 + i * CH, CH)])
    pltpu.sync_copy(u, uloc)

    cpd = jnp.where(cid == 0, SPB0, SPB1)
    c0 = jnp.where(cid == 0, sid * SPB0, NS * SPB0 + sid * SPB1)
    plsc.subcore_barrier()

    pltpu.async_copy(srcm.at[c0], srcba, semsa)
    pltpu.async_copy(dstm.at[c0], dstba, semda)
    pltpu.async_copy(srcm.at[c0 + 1], srcbb, semsb)
    pltpu.async_copy(dstm.at[c0 + 1], dstbb, semdb)

    def _gather(srcb, valb):
        def _gs(k, carry):
            sv = srcb[pl.ds(k * 16, 16)]
            valb[pl.ds(k * 16, 16)] = plsc.load_gather(uloc, [sv])
            return carry
        lax.fori_loop(0, CH // 16, _gs, 0)

    def _step(j, carry):
        ia = 2 * j
        ib = 2 * j + 1
        pltpu.make_async_copy(srcm.at[c0 + ia], srcba, semsa).wait()
        _gather(srcba, valba)
        pltpu.make_async_copy(dstm.at[c0 + ia], dstba, semda).wait()
        pltpu.async_copy(valba, acc.at[dstba], semca, add=True)
        pltpu.make_async_copy(srcm.at[c0 + ib], srcbb, semsb).wait()
        _gather(srcbb, valbb)
        pltpu.make_async_copy(valba, acc.at[dstba], semca).wait()

        @pl.when(ia + 2 < cpd)
        def _():
            pltpu.async_copy(srcm.at[c0 + ia + 2], srcba, semsa)
            pltpu.async_copy(dstm.at[c0 + ia + 2], dstba, semda)

        pltpu.make_async_copy(dstm.at[c0 + ib], dstbb, semdb).wait()
        pltpu.async_copy(valbb, acc.at[dstbb], semcb, add=True)
        pltpu.make_async_copy(valbb, acc.at[dstbb], semcb).wait()

        @pl.when(ib + 2 < cpd)
        def _():
            pltpu.async_copy(srcm.at[c0 + ib + 2], srcbb, semsb)
            pltpu.async_copy(dstm.at[c0 + ib + 2], dstbb, semdb)

        return carry

    lax.fori_loop(0, cpd // 2, _step, 0)
    plsc.subcore_barrier()

    @pl.when(cid == 0)
    def _():
        pltpu.sync_copy(acc.at[pl.ds(rows0, rpt)], out0.at[pl.ds(rows0, rpt)])

    @pl.when(cid == 1)
    def _():
        pltpu.sync_copy(acc.at[pl.ds(rows0, rpt)], out1.at[pl.ds(rows0, rpt)])


def _tc_cntsum_body(cp, o):
    o[...] = jnp.sum(cp[...], axis=0)


def _tc_dense_body(s2, c3, xb, w1l, w1r, b1, w2l, w2r, wc, b2, bcb,
                   u_o, v_o, cm_o):
    s = s2[...]
    cntm = jnp.maximum(c3[...], 1.0)
    mean = s / cntm
    h = jnp.dot(mean, w1l[...], preferred_element_type=jnp.float32)
    h = h + jnp.dot(xb[...], w1r[...], preferred_element_type=jnp.float32)
    h = jnp.maximum(h + b1[...], 0.0)
    w2 = jnp.dot(w2l[...], wc[...], preferred_element_type=jnp.float32)
    wr = jnp.dot(w2r[...], wc[...], preferred_element_type=jnp.float32)
    c0s = jnp.dot(b2[...], wc[...], preferred_element_type=jnp.float32)
    u_o[...] = jnp.dot(h, w2, preferred_element_type=jnp.float32)
    v_o[...] = jnp.dot(h, wr, preferred_element_type=jnp.float32) + c0s + bcb[...]
    cm_o[...] = cntm


def _tc_final_body(p0, p1, cm, vb, o):
    o[...] = (p0[...] + p1[...]) / cm[...] + vb[...]


def kernel(x, edge_index, W1_l, b1_l, W1_r, W2_l, b2_l, W2_r, Wc, bc):
    n, d = x.shape
    e = edge_index.shape[1]
    n_pad = (-(-n // CH) + 1) * CH
    n_pad = -(-n_pad // (NS * CH)) * NS * CH
    e_pad = NS * CPBT * CH

    f32 = jnp.float32
    xp = jnp.pad(x, ((0, n_pad - n), (0, 0)))
    pad = jnp.full((e_pad - e,), n, jnp.int32)
    srcm = jnp.concatenate([edge_index[0], pad]).reshape(e_pad // CH, CH)
    dstm = jnp.concatenate([edge_index[1], pad]).reshape(e_pad // CH, CH)

    mesh = plsc.VectorSubcoreMesh(core_axis_name="c", subcore_axis_name="s")
    mesh1 = plsc.VectorSubcoreMesh(core_axis_name="c", subcore_axis_name="s",
                                   num_cores=1)

    agg_rows = pl.kernel(
        functools.partial(_sc_agg_rows_body, n_pad, d),
        out_type=[jax.ShapeDtypeStruct((n_pad, d), f32),
                  jax.ShapeDtypeStruct((NS, n_pad // CH, CH), f32)],
        mesh=mesh1,
        scratch_types=[
            pltpu.VMEM((CH,), jnp.int32),
            pltpu.VMEM((CH,), jnp.int32),
            pltpu.VMEM((CH,), jnp.int32),
            pltpu.VMEM((CH,), jnp.int32),
            pltpu.VMEM((CH, d), f32),
            pltpu.VMEM((CH, d), f32),
            pltpu.VMEM((n_pad // CH, CH), f32),
            pltpu.SemaphoreType.DMA,
            pltpu.SemaphoreType.DMA,
            pltpu.SemaphoreType.DMA,
            pltpu.SemaphoreType.DMA,
            pltpu.SemaphoreType.DMA,
            pltpu.SemaphoreType.DMA,
            pltpu.SemaphoreType.DMA,
            pltpu.SemaphoreType.DMA,
            pltpu.VMEM_SHARED((n_pad, d), f32),
        ],
        compiler_params=pltpu.CompilerParams(needs_layout_passes=False),
    )
    s_part, cnt_part = agg_rows(xp, srcm, dstm)

    rows = n_pad // CH
    cnt2 = pl.pallas_call(
        _tc_cntsum_body,
        grid=(1,),
        in_specs=[pl.BlockSpec((NS, rows, CH), lambda i: (0, 0, 0))],
        out_specs=pl.BlockSpec((rows, CH), lambda i: (0, 0)),
        out_shape=jax.ShapeDtypeStruct((rows, CH), f32),
    )(cnt_part)

    grid_r = 1024
    gsteps = n_pad // grid_r
    u, v, cm = pl.pallas_call(
        _tc_dense_body,
        grid=(gsteps,),
        in_specs=[
            pl.BlockSpec((grid_r, d), lambda i: (i, 0)),
            pl.BlockSpec((grid_r, 1), lambda i: (i, 0)),
            pl.BlockSpec((grid_r, d), lambda i: (i, 0)),
            pl.BlockSpec((d, d), lambda i: (0, 0)),
            pl.BlockSpec((d, d), lambda i: (0, 0)),
            pl.BlockSpec((1, d), lambda i: (0, 0)),
            pl.BlockSpec((d, d), lambda i: (0, 0)),
            pl.BlockSpec((d, d), lambda i: (0, 0)),
            pl.BlockSpec((d, 1), lambda i: (0, 0)),
            pl.BlockSpec((1, d), lambda i: (0, 0)),
            pl.BlockSpec((1, 1), lambda i: (0, 0)),
        ],
        out_specs=[
            pl.BlockSpec((grid_r, 1), lambda i: (i, 0)),
            pl.BlockSpec((grid_r, 1), lambda i: (i, 0)),
            pl.BlockSpec((grid_r, 1), lambda i: (i, 0)),
        ],
        out_shape=[
            jax.ShapeDtypeStruct((n_pad, 1), f32),
            jax.ShapeDtypeStruct((n_pad, 1), f32),
            jax.ShapeDtypeStruct((n_pad, 1), f32),
        ],
        compiler_params=pltpu.CompilerParams(
            dimension_semantics=("arbitrary",)),
    )(s_part, cnt2.reshape(n_pad, 1), xp, W1_l, W1_r,
      b1_l.reshape(1, d), W2_l, W2_r, Wc, b2_l.reshape(1, d),
      bc.reshape(1, 1))

    agg_scalar = pl.kernel(
        functools.partial(_sc_agg_scalar_body, n_pad),
        out_type=[jax.ShapeDtypeStruct((n_pad,), f32),
                  jax.ShapeDtypeStruct((n_pad,), f32)],
        mesh=mesh,
        scratch_types=[
            pltpu.VMEM((n_pad,), f32),
            pltpu.VMEM((CH,), jnp.int32),
            pltpu.VMEM((CH,), jnp.int32),
            pltpu.VMEM((CH,), jnp.int32),
            pltpu.VMEM((CH,), jnp.int32),
            pltpu.VMEM((CH,), f32),
            pltpu.VMEM((CH,), f32),
            pltpu.SemaphoreType.DMA,
            pltpu.SemaphoreType.DMA,
            pltpu.SemaphoreType.DMA,
            pltpu.SemaphoreType.DMA,
            pltpu.SemaphoreType.DMA,
            pltpu.SemaphoreType.DMA,
            pltpu.VMEM_SHARED((n_pad,), f32),
        ],
        compiler_params=pltpu.CompilerParams(needs_layout_passes=False),
    )
    p0, p1 = agg_scalar(u.reshape(n_pad), srcm, dstm)
    out2d = pl.pallas_call(
        _tc_final_body,
        grid=(1,),
        in_specs=[pl.BlockSpec((rows, CH), lambda i: (0, 0))] * 4,
        out_specs=pl.BlockSpec((rows, CH), lambda i: (0, 0)),
        out_shape=jax.ShapeDtypeStruct((rows, CH), f32),
    )(p0.reshape(rows, CH), p1.reshape(rows, CH),
      cm.reshape(rows, CH), v.reshape(rows, CH))
    return out2d.reshape(n_pad)[:n]


# speedup vs baseline: 1.3709x; 1.3709x over previous
import functools

import jax
import jax.numpy as jnp
from jax import lax
from jax.experimental import pallas as pl
from jax.experimental.pallas import tpu as pltpu
from jax.experimental.pallas import tpu_sc as plsc

NC = 2
NS = 16
NW = NC * NS
CH = 128
CPB0 = 144
CPB1 = 16
SPB0 = 80
SPB1 = 80


def _sc_agg_rows_body(n_pad, d, xp, srcm, dstm,
                      out_s, out_cnt,
                      srcba, srcbb, dstba, dstbb, bufa, bufb, cnt2d,
                      semsa, semsb, semda, semdb, semga, semgb, semca, semcb,
                      acc):
    cid = lax.axis_index("c")
    sid = lax.axis_index("s")
    rpt = n_pad // NS
    rows0 = sid * rpt

    def _zb(t, carry):
        bufa[t >> 3, pl.ds((t & 7) * 16, 16)] = jnp.zeros((16,), jnp.float32)
        return carry
    lax.fori_loop(0, CH * (d // 16), _zb, 0)
    for i in range(rpt // CH):
        pltpu.sync_copy(bufa, acc.at[pl.ds(rows0 + i * CH, CH)])

    def _zc(t, carry):
        cnt2d[t >> 3, pl.ds((t & 7) * 16, 16)] = jnp.zeros((16,), jnp.float32)
        return carry
    lax.fori_loop(0, (n_pad // CH) * (CH // 16), _zc, 0)

    cpb = jnp.where(cid == 0, CPB0, CPB1)
    c0 = jnp.where(cid == 0, sid * CPB0, NS * CPB0 + sid * CPB1)
    plsc.subcore_barrier()

    pltpu.async_copy(srcm.at[c0], srcba, semsa)
    pltpu.async_copy(dstm.at[c0], dstba, semda)
    pltpu.async_copy(srcm.at[c0 + 1], srcbb, semsb)
    pltpu.async_copy(dstm.at[c0 + 1], dstbb, semdb)
    pltpu.make_async_copy(srcm.at[c0], srcba, semsa).wait()
    pltpu.async_copy(xp.at[srcba], bufa, semga)

    def _count(dstb):
        def _cs(k, carry):
            dv = dstb[pl.ds(k * 16, 16)]
            occ, last = plsc.scan_count(dv)
            vals = occ.astype(jnp.float32)
            plsc.addupdate_scatter(
                cnt2d,
                [lax.shift_right_logical(dv, 7), lax.bitwise_and(dv, 127)],
                vals, mask=last)
            return carry
        lax.fori_loop(0, CH // 16, _cs, 0)

    def _step(j, carry):
        ia = 2 * j
        ib = 2 * j + 1
        pltpu.make_async_copy(srcm.at[c0 + ib], srcbb, semsb).wait()
        pltpu.async_copy(xp.at[srcbb], bufb, semgb)
        pltpu.make_async_copy(xp.at[srcba], bufa, semga).wait()
        pltpu.make_async_copy(dstm.at[c0 + ia], dstba, semda).wait()
        pltpu.async_copy(bufa, acc.at[dstba], semca, add=True)
        _count(dstba)
        pltpu.make_async_copy(bufa, acc.at[dstba], semca).wait()

        @pl.when(ia + 2 < cpb)
        def _():
            pltpu.async_copy(srcm.at[c0 + ia + 2], srcba, semsa)
            pltpu.async_copy(dstm.at[c0 + ia + 2], dstba, semda)
            pltpu.make_async_copy(srcm.at[c0 + ia + 2], srcba, semsa).wait()
            pltpu.async_copy(xp.at[srcba], bufa, semga)

        pltpu.make_async_copy(xp.at[srcbb], bufb, semgb).wait()
        pltpu.make_async_copy(dstm.at[c0 + ib], dstbb, semdb).wait()
        pltpu.async_copy(bufb, acc.at[dstbb], semcb, add=True)
        _count(dstbb)
        pltpu.make_async_copy(bufb, acc.at[dstbb], semcb).wait()

        @pl.when(ib + 2 < cpb)
        def _():
            pltpu.async_copy(srcm.at[c0 + ib + 2], srcbb, semsb)
            pltpu.async_copy(dstm.at[c0 + ib + 2], dstbb, semdb)

        return carry

    lax.fori_loop(0, cpb // 2, _step, 0)
    plsc.subcore_barrier()

    for i in range(rpt // CH):
        buf = bufa if i % 2 == 0 else bufb
        pltpu.sync_copy(acc.at[pl.ds(rows0 + i * CH, CH)], buf)
        pltpu.sync_copy(buf, out_s.at[cid, pl.ds(rows0 + i * CH, CH)])
    pltpu.sync_copy(cnt2d, out_cnt.at[cid * NS + sid])


def _sc_agg_scalar_body(n_pad, u, srcm, dstm, out0, out1,
                        uloc, srcba, srcbb, dstba, dstbb, valba, valbb,
                        semsa, semsb, semda, semdb, semca, semcb, acc):
    cid = lax.axis_index("c")
    sid = lax.axis_index("s")
    rpt = n_pad // NS
    rows0 = sid * rpt

    def _zv(k, carry):
        valba[pl.ds(k * 16, 16)] = jnp.zeros((16,), jnp.float32)
        return carry
    lax.fori_loop(0, CH // 16, _zv, 0)
    for i in range(rpt // CH):
        pltpu.sync_copy(valba, acc.at[pl.ds(rows0 + i * CH, CH)])
    pltpu.sync_copy(u, uloc)

    cpd = jnp.where(cid == 0, SPB0, SPB1)
    c0 = jnp.where(cid == 0, sid * SPB0, NS * SPB0 + sid * SPB1)
    plsc.subcore_barrier()

    pltpu.async_copy(srcm.at[c0], srcba, semsa)
    pltpu.async_copy(dstm.at[c0], dstba, semda)
    pltpu.async_copy(srcm.at[c0 + 1], srcbb, semsb)
    pltpu.async_copy(dstm.at[c0 + 1], dstbb, semdb)

    def _gather(srcb, valb):
        def _gs(k, carry):
            sv = srcb[pl.ds(k * 16, 16)]
            valb[pl.ds(k * 16, 16)] = plsc.load_gather(uloc, [sv])
            return carry
        lax.fori_loop(0, CH // 16, _gs, 0)

    def _step(j, carry):
        ia = 2 * j
        ib = 2 * j + 1
        pltpu.make_async_copy(srcm.at[c0 + ia], srcba, semsa).wait()
        _gather(srcba, valba)
        pltpu.make_async_copy(dstm.at[c0 + ia], dstba, semda).wait()
        pltpu.async_copy(valba, acc.at[dstba], semca, add=True)
        pltpu.make_async_copy(srcm.at[c0 + ib], srcbb, semsb).wait()
        _gather(srcbb, valbb)
        pltpu.make_async_copy(valba, acc.at[dstba], semca).wait()

        @pl.when(ia + 2 < cpd)
        def _():
            pltpu.async_copy(srcm.at[c0 + ia + 2], srcba, semsa)
            pltpu.async_copy(dstm.at[c0 + ia + 2], dstba, semda)

        pltpu.make_async_copy(dstm.at[c0 + ib], dstbb, semdb).wait()
        pltpu.async_copy(valbb, acc.at[dstbb], semcb, add=True)
        pltpu.make_async_copy(valbb, acc.at[dstbb], semcb).wait()

        @pl.when(ib + 2 < cpd)
        def _():
            pltpu.async_copy(srcm.at[c0 + ib + 2], srcbb, semsb)
            pltpu.async_copy(dstm.at[c0 + ib + 2], dstbb, semdb)

        return carry

    lax.fori_loop(0, cpd // 2, _step, 0)
    plsc.subcore_barrier()

    @pl.when(cid == 0)
    def _():
        pltpu.sync_copy(acc.at[pl.ds(rows0, rpt)], out0.at[pl.ds(rows0, rpt)])

    @pl.when(cid == 1)
    def _():
        pltpu.sync_copy(acc.at[pl.ds(rows0, rpt)], out1.at[pl.ds(rows0, rpt)])


def _tc_cntsum_body(cp, o):
    o[...] = jnp.sum(cp[...], axis=0)


def _tc_dense_body(s2, c3, xb, w1l, w1r, b1, w2l, w2r, wc, b2, bcb,
                   u_o, v_o, cm_o):
    s = s2[0] + s2[1]
    cntm = jnp.maximum(c3[...], 1.0)
    mean = s / cntm
    h = jnp.dot(mean, w1l[...], preferred_element_type=jnp.float32)
    h = h + jnp.dot(xb[...], w1r[...], preferred_element_type=jnp.float32)
    h = jnp.maximum(h + b1[...], 0.0)
    w2 = jnp.dot(w2l[...], wc[...], preferred_element_type=jnp.float32)
    wr = jnp.dot(w2r[...], wc[...], preferred_element_type=jnp.float32)
    c0s = jnp.dot(b2[...], wc[...], preferred_element_type=jnp.float32)
    u_o[...] = jnp.dot(h, w2, preferred_element_type=jnp.float32)
    v_o[...] = jnp.dot(h, wr, preferred_element_type=jnp.float32) + c0s + bcb[...]
    cm_o[...] = cntm


def _tc_final_body(p0, p1, cm, vb, o):
    o[...] = (p0[...] + p1[...]) / cm[...] + vb[...]


def kernel(x, edge_index, W1_l, b1_l, W1_r, W2_l, b2_l, W2_r, Wc, bc):
    n, d = x.shape
    e = edge_index.shape[1]
    n_pad = (-(-n // CH) + 1) * CH
    n_pad = -(-n_pad // (NS * CH)) * NS * CH
    e_pad = NS * (CPB0 + CPB1) * CH

    f32 = jnp.float32
    xp = jnp.pad(x, ((0, n_pad - n), (0, 0)))
    pad = jnp.full((e_pad - e,), n, jnp.int32)
    srcm = jnp.concatenate([edge_index[0], pad]).reshape(e_pad // CH, CH)
    dstm = jnp.concatenate([edge_index[1], pad]).reshape(e_pad // CH, CH)

    mesh = plsc.VectorSubcoreMesh(core_axis_name="c", subcore_axis_name="s")
    mesh1 = plsc.VectorSubcoreMesh(core_axis_name="c", subcore_axis_name="s",
                                   num_cores=1)

    agg_rows = pl.kernel(
        functools.partial(_sc_agg_rows_body, n_pad, d),
        out_type=[jax.ShapeDtypeStruct((NC, n_pad, d), f32),
                  jax.ShapeDtypeStruct((NW, n_pad // CH, CH), f32)],
        mesh=mesh,
        scratch_types=[
            pltpu.VMEM((CH,), jnp.int32),
            pltpu.VMEM((CH,), jnp.int32),
            pltpu.VMEM((CH,), jnp.int32),
            pltpu.VMEM((CH,), jnp.int32),
            pltpu.VMEM((CH, d), f32),
            pltpu.VMEM((CH, d), f32),
            pltpu.VMEM((n_pad // CH, CH), f32),
            pltpu.SemaphoreType.DMA,
            pltpu.SemaphoreType.DMA,
            pltpu.SemaphoreType.DMA,
            pltpu.SemaphoreType.DMA,
            pltpu.SemaphoreType.DMA,
            pltpu.SemaphoreType.DMA,
            pltpu.SemaphoreType.DMA,
            pltpu.SemaphoreType.DMA,
            pltpu.VMEM_SHARED((n_pad, d), f32),
        ],
        compiler_params=pltpu.CompilerParams(needs_layout_passes=False),
    )
    s_part, cnt_part = agg_rows(xp, srcm, dstm)

    rows = n_pad // CH
    cnt2 = pl.pallas_call(
        _tc_cntsum_body,
        grid=(1,),
        in_specs=[pl.BlockSpec((NW, rows, CH), lambda i: (0, 0, 0))],
        out_specs=pl.BlockSpec((rows, CH), lambda i: (0, 0)),
        out_shape=jax.ShapeDtypeStruct((rows, CH), f32),
    )(cnt_part)

    grid_r = 1024
    gsteps = n_pad // grid_r
    u, v, cm = pl.pallas_call(
        _tc_dense_body,
        grid=(gsteps,),
        in_specs=[
            pl.BlockSpec((NC, grid_r, d), lambda i: (0, i, 0)),
            pl.BlockSpec((grid_r, 1), lambda i: (i, 0)),
            pl.BlockSpec((grid_r, d), lambda i: (i, 0)),
            pl.BlockSpec((d, d), lambda i: (0, 0)),
            pl.BlockSpec((d, d), lambda i: (0, 0)),
            pl.BlockSpec((1, d), lambda i: (0, 0)),
            pl.BlockSpec((d, d), lambda i: (0, 0)),
            pl.BlockSpec((d, d), lambda i: (0, 0)),
            pl.BlockSpec((d, 1), lambda i: (0, 0)),
            pl.BlockSpec((1, d), lambda i: (0, 0)),
            pl.BlockSpec((1, 1), lambda i: (0, 0)),
        ],
        out_specs=[
            pl.BlockSpec((grid_r, 1), lambda i: (i, 0)),
            pl.BlockSpec((grid_r, 1), lambda i: (i, 0)),
            pl.BlockSpec((grid_r, 1), lambda i: (i, 0)),
        ],
        out_shape=[
            jax.ShapeDtypeStruct((n_pad, 1), f32),
            jax.ShapeDtypeStruct((n_pad, 1), f32),
            jax.ShapeDtypeStruct((n_pad, 1), f32),
        ],
        compiler_params=pltpu.CompilerParams(
            dimension_semantics=("arbitrary",)),
    )(s_part, cnt2.reshape(n_pad, 1), xp, W1_l, W1_r,
      b1_l.reshape(1, d), W2_l, W2_r, Wc, b2_l.reshape(1, d),
      bc.reshape(1, 1))

    agg_scalar = pl.kernel(
        functools.partial(_sc_agg_scalar_body, n_pad),
        out_type=[jax.ShapeDtypeStruct((n_pad,), f32),
                  jax.ShapeDtypeStruct((n_pad,), f32)],
        mesh=mesh,
        scratch_types=[
            pltpu.VMEM((n_pad,), f32),
            pltpu.VMEM((CH,), jnp.int32),
            pltpu.VMEM((CH,), jnp.int32),
            pltpu.VMEM((CH,), jnp.int32),
            pltpu.VMEM((CH,), jnp.int32),
            pltpu.VMEM((CH,), f32),
            pltpu.VMEM((CH,), f32),
            pltpu.SemaphoreType.DMA,
            pltpu.SemaphoreType.DMA,
            pltpu.SemaphoreType.DMA,
            pltpu.SemaphoreType.DMA,
            pltpu.SemaphoreType.DMA,
            pltpu.SemaphoreType.DMA,
            pltpu.VMEM_SHARED((n_pad,), f32),
        ],
        compiler_params=pltpu.CompilerParams(needs_layout_passes=False),
    )
    p0, p1 = agg_scalar(u.reshape(n_pad), srcm, dstm)
    out2d = pl.pallas_call(
        _tc_final_body,
        grid=(1,),
        in_specs=[pl.BlockSpec((rows, CH), lambda i: (0, 0))] * 4,
        out_specs=pl.BlockSpec((rows, CH), lambda i: (0, 0)),
        out_shape=jax.ShapeDtypeStruct((rows, CH), f32),
    )(p0.reshape(rows, CH), p1.reshape(rows, CH),
      cm.reshape(rows, CH), v.reshape(rows, CH))
    return out2d.reshape(n_pad)[:n]
